# baseline (speedup 1.0000x reference)
import functools

import jax
import jax.numpy as jnp
from jax import lax
from jax.experimental import pallas as pl
from jax.experimental.pallas import tpu as pltpu
from jax.experimental.pallas import tpu_sc as plsc

_K = 64
_NS = 16
_NC = 2
_L = 16
_NB = 1024
_R0, _R1, _R2 = 0, _L * _NB, 2 * _L * _NB


def _find_bucket(merged, csum, rem):
    nchunk = _NB // _L

    def fc_body(t, carry):
        acc, sel, selacc = carry
        ci = nchunk - 1 - t
        sv = csum[ci]
        hit = (sel < 0) & (acc + sv >= rem)
        sel = jnp.where(hit, ci, sel)
        selacc = jnp.where(hit, acc, selacc)
        return acc + sv, sel, selacc

    _, sel, selacc = lax.fori_loop(
        0, nchunk, fc_body,
        (jnp.int32(0), jnp.int32(-1), jnp.int32(0)))

    chunkv = merged[pl.ds(sel * _L, _L)]
    b_star = jnp.int32(-1)
    above = jnp.int32(0)
    acc = selacc
    for l in range(_L - 1, -1, -1):
        sv = chunkv[l]
        hit = (b_star < 0) & (acc + sv >= rem)
        b_star = jnp.where(hit, sel * _L + l, b_star)
        above = jnp.where(hit, acc, above)
        acc = acc + sv
    return b_star, above


def _sc_body(x_hbm, out_hbm, ubuf, hist, merged, csum,
             sh0, sh1, sh2, shcnt, dmasem, *, n, kk):
    seg = n // _NS
    half = seg // _NC
    nvec = seg // _L
    cid = lax.axis_index("c")
    sid = lax.axis_index("s")
    base = sid * seg
    lane = lax.iota(jnp.int32, _L)
    ones = jnp.ones((_L,), jnp.int32)
    zeros16 = jnp.zeros((_L,), jnp.int32)

    cp = pltpu.async_copy(x_hbm.at[pl.ds(base, seg)], ubuf, dmasem)

    @plsc.parallel_loop(0, 3 * _NB, unroll=8)
    def _(j):
        hist[pl.ds(j * _L, _L)] = zeros16

    cp.wait()

    def lane_reduce(region, with_sums=False):
        @plsc.parallel_loop(0, _NB // _L, unroll=2)
        def _(j):
            acc = hist[pl.ds(region + j * _L, _L)]
            for l in range(1, _L):
                acc = acc + hist[pl.ds(region + l * _NB + j * _L, _L)]
            merged[pl.ds(j * _L, _L)] = acc
            if with_sums:
                csum[j] = jnp.sum(acc)

    def merge_round(sh, region):
        lane_reduce(region)
        pltpu.sync_copy(merged.at[pl.ds(0, _NB)], sh.at[pl.ds(sid * _NB, _NB)])
        plsc.subcore_barrier()
        pltpu.sync_copy(sh, hist.at[pl.ds(region, _NS * _NB)])
        lane_reduce(region, with_sums=True)

    @plsc.parallel_loop(0, nvec, unroll=8)
    def _(i):
        u = jnp.maximum(ubuf[pl.ds(i * _L, _L)], 0)
        b = lax.shift_right_logical(u, 21)
        plsc.addupdate_scatter(hist, [_R0 + lane * _NB + b], ones)

    merge_round(sh0, _R0)
    b0, above0 = _find_bucket(merged, csum, jnp.int32(kk))
    rem1 = kk - above0

    @plsc.parallel_loop(0, nvec, unroll=8)
    def _(i):
        u = jnp.maximum(ubuf[pl.ds(i * _L, _L)], 0)
        m = lax.shift_right_logical(u, 21) == b0
        b = lax.shift_right_logical(u, 11) & (_NB - 1)
        plsc.addupdate_scatter(hist, [_R1 + lane * _NB + b], ones, mask=m)

    merge_round(sh1, _R1)
    b1, above1 = _find_bucket(merged, csum, rem1)
    p2 = (b0 << 10) | b1
    rem2 = rem1 - above1

    @plsc.parallel_loop(0, nvec, unroll=8)
    def _(i):
        u = jnp.maximum(ubuf[pl.ds(i * _L, _L)], 0)
        m = lax.shift_right_logical(u, 11) == p2
        b = lax.shift_right_logical(u, 1) & (_NB - 1)
        plsc.addupdate_scatter(hist, [_R2 + lane * _NB + b], ones, mask=m)

    merge_round(sh2, _R2)
    b2, above2 = _find_bucket(merged, csum, rem2)
    t0_bits = (((p2 << 10) | b2) << 1)
    t1_bits = t0_bits | 1
    rem3 = rem2 - above2

    def cnt_body(off):
        @plsc.parallel_loop(0, half // _L, unroll=8, carry=(zeros16, zeros16))
        def accs(i, carry):
            a0, a1 = carry
            u = jnp.maximum(ubuf[pl.ds(off + i * _L, _L)], 0)
            return (a0 + (u == t0_bits).astype(jnp.int32),
                    a1 + (u == t1_bits).astype(jnp.int32))
        a0, a1 = accs
        return jnp.sum(a0), jnp.sum(a1)

    n0_t0, n0_t1 = cnt_body(0)
    n1_t0, n1_t1 = cnt_body(half)
    row = jnp.where(lane == 0, n0_t0,
                    jnp.where(lane == 1, n1_t0,
                              jnp.where(lane == 2, n0_t1,
                                        jnp.where(lane == 3, n1_t1, 0))))
    merged[pl.ds(0, _L)] = row
    pltpu.sync_copy(merged.at[pl.ds(0, _L)], shcnt.at[pl.ds(sid * _L, _L)])
    plsc.subcore_barrier()
    pltpu.sync_copy(shcnt, hist.at[pl.ds(0, _NS * _L)])

    c_t1 = jnp.int32(0)
    for sp in range(_NS):
        rowv = hist[pl.ds(sp * _L, _L)]
        c_t1 = c_t1 + rowv[2] + rowv[3]
    use_t1 = rem3 <= c_t1
    t_bits = jnp.where(use_t1, t1_bits, t0_bits)
    e_keep = jnp.where(use_t1, rem3, rem3 - c_t1)

    h_own = 2 * sid + cid
    pre = jnp.int32(0)
    for sp in range(_NS):
        rowv = hist[pl.ds(sp * _L, _L)]
        a = jnp.where(use_t1, rowv[2], rowv[0])
        b = jnp.where(use_t1, rowv[3], rowv[1])
        pre = pre + jnp.where(2 * sp < h_own, a, 0)
        pre = pre + jnp.where(2 * sp + 1 < h_own, b, 0)
    n_own = jnp.where(cid == 0,
                      jnp.where(use_t1, n0_t1, n0_t0),
                      jnp.where(use_t1, n1_t1, n1_t0))
    quota = jnp.clip(e_keep - pre, 0, n_own)

    off = cid * half

    @pl.when(quota == n_own)
    def _():
        @plsc.parallel_loop(0, half // _L, unroll=8)
        def _(i):
            u = jnp.maximum(ubuf[pl.ds(off + i * _L, _L)], 0)
            ubuf[pl.ds(off + i * _L, _L)] = jnp.where(u >= t_bits, u, 0)

    @pl.when(jnp.logical_and(quota == 0, n_own > 0))
    def _():
        @plsc.parallel_loop(0, half // _L, unroll=8)
        def _(i):
            u = jnp.maximum(ubuf[pl.ds(off + i * _L, _L)], 0)
            ubuf[pl.ds(off + i * _L, _L)] = jnp.where(u > t_bits, u, 0)

    @pl.when(jnp.logical_and(quota > 0, quota < n_own))
    def _():
        def sb(i, running):
            u = jnp.maximum(ubuf[pl.ds(off + i * _L, _L)], 0)
            eq = (u == t_bits).astype(jnp.int32)
            csum_v = plsc.cumsum(eq)
            rank = running + csum_v - eq
            keep = (u > t_bits) | ((eq == 1) & (rank < quota))
            ubuf[pl.ds(off + i * _L, _L)] = jnp.where(keep, u, 0)
            return running + csum_v[_L - 1]
        lax.fori_loop(0, half // _L, sb, jnp.int32(0))

    pltpu.sync_copy(ubuf.at[pl.ds(off, half)],
                    out_hbm.at[pl.ds(base + off, half)])


def _make_sc_call(n, kk):
    seg = n // _NS
    mesh = plsc.VectorSubcoreMesh(core_axis_name="c", subcore_axis_name="s",
                                  num_cores=_NC, num_subcores=_NS)
    return pl.kernel(
        functools.partial(_sc_body, n=n, kk=kk),
        out_type=jax.ShapeDtypeStruct((n,), jnp.int32),
        mesh=mesh,
        compiler_params=pltpu.CompilerParams(needs_layout_passes=False),
        scratch_types=[
            pltpu.VMEM((seg,), jnp.int32),
            pltpu.VMEM((3 * _L * _NB,), jnp.int32),
            pltpu.VMEM((_NB,), jnp.int32),
            pltpu.SMEM((_NB // _L,), jnp.int32),
            pltpu.VMEM_SHARED((_NS * _NB,), jnp.int32),
            pltpu.VMEM_SHARED((_NS * _NB,), jnp.int32),
            pltpu.VMEM_SHARED((_NS * _NB,), jnp.int32),
            pltpu.VMEM_SHARED((_NS * _L,), jnp.int32),
            pltpu.SemaphoreType.DMA,
        ],
    )


def kernel(x):
    n = x.size
    kk = _K * x.shape[0]
    xi = lax.bitcast_convert_type(x.reshape(-1), jnp.int32)
    out = _make_sc_call(n, kk)(xi)
    return lax.bitcast_convert_type(out, jnp.float32).reshape(x.shape)


# speedup vs baseline: 1.1100x; 1.1100x over previous
import functools

import jax
import jax.numpy as jnp
from jax import lax
from jax.experimental import pallas as pl
from jax.experimental.pallas import tpu as pltpu
from jax.experimental.pallas import tpu_sc as plsc

_K = 64
_NS = 16
_NC = 2
_L = 16
_NB = 1024
_R0, _R1, _R2 = 0, _L * _NB, 2 * _L * _NB


def _find_bucket(merged, csum, rem):
    nchunk = _NB // _L

    def fc_body(t, carry):
        acc, sel, selacc = carry
        ci = nchunk - 1 - t
        sv = csum[ci]
        hit = (sel < 0) & (acc + sv >= rem)
        sel = jnp.where(hit, ci, sel)
        selacc = jnp.where(hit, acc, selacc)
        return acc + sv, sel, selacc

    _, sel, selacc = lax.fori_loop(
        0, nchunk, fc_body,
        (jnp.int32(0), jnp.int32(-1), jnp.int32(0)))

    chunkv = merged[pl.ds(sel * _L, _L)]
    b_star = jnp.int32(-1)
    above = jnp.int32(0)
    acc = selacc
    for l in range(_L - 1, -1, -1):
        sv = chunkv[l]
        hit = (b_star < 0) & (acc + sv >= rem)
        b_star = jnp.where(hit, sel * _L + l, b_star)
        above = jnp.where(hit, acc, above)
        acc = acc + sv
    return b_star, above


def _sc_body(x_hbm, out_hbm, ubuf, hist, merged, cand, csum,
             sh0, sh1, sh2, shcnt, dmasem, *, n, kk):
    seg = n // _NS
    half = seg // _NC
    nvec = seg // _L
    cid = lax.axis_index("c")
    sid = lax.axis_index("s")
    base = sid * seg
    lane = lax.iota(jnp.int32, _L)
    ones = jnp.ones((_L,), jnp.int32)
    zeros16 = jnp.zeros((_L,), jnp.int32)

    cp = pltpu.async_copy(x_hbm.at[pl.ds(base, seg)], ubuf, dmasem)

    @plsc.parallel_loop(0, 3 * _NB, unroll=8)
    def _(j):
        hist[pl.ds(j * _L, _L)] = zeros16

    cp.wait()

    def lane_reduce(region, with_sums=False):
        @plsc.parallel_loop(0, _NB // _L, unroll=2)
        def _(j):
            acc = hist[pl.ds(region + j * _L, _L)]
            for l in range(1, _L):
                acc = acc + hist[pl.ds(region + l * _NB + j * _L, _L)]
            merged[pl.ds(j * _L, _L)] = acc
            if with_sums:
                csum[j] = jnp.sum(acc)

    def merge_round(sh, region):
        lane_reduce(region)
        pltpu.sync_copy(merged.at[pl.ds(0, _NB)], sh.at[pl.ds(sid * _NB, _NB)])
        plsc.subcore_barrier()
        pltpu.sync_copy(sh, hist.at[pl.ds(region, _NS * _NB)])
        lane_reduce(region, with_sums=True)

    @plsc.parallel_loop(0, nvec, unroll=8)
    def _(i):
        w = ubuf[pl.ds(i * _L, _L)]
        u = jnp.maximum(w, 0)
        ubuf[pl.ds(i * _L, _L)] = u
        b = lax.shift_right_logical(u, 21)
        plsc.addupdate_scatter(hist, [_R0 + lane * _NB + b], ones)

    merge_round(sh0, _R0)
    b0, above0 = _find_bucket(merged, csum, jnp.int32(kk))
    rem1 = kk - above0

    neg1 = jnp.full((_L,), -1, jnp.int32)
    c1_base = half + _L

    def compact(lo, hi, o_init):
        @plsc.parallel_loop(lo, hi, unroll=4, carry=jnp.int32(o_init))
        def o_fin(i, o):
            u = ubuf[pl.ds(i * _L, _L)]
            m = lax.shift_right_logical(u, 21) == b0
            plsc.store_compressed(cand.at[pl.ds(o, _L)], u, mask=m)
            return o + plsc.all_reduce_population_count(m)[0]
        cand[pl.ds(o_fin, _L)] = neg1
        return o_fin

    o0 = compact(0, half // _L, 0)
    o1 = compact(half // _L, nvec, c1_base)
    nv0 = (o0 + _L - 1) // _L
    nv1 = (o1 - c1_base + _L - 1) // _L

    def cand_scan(nv, cbase, body):
        @plsc.parallel_loop(0, nv, unroll=2)
        def _(i):
            body(cand[pl.ds(cbase + i * _L, _L)])

    def r1_body(u):
        m = lax.shift_right_logical(u, 21) == b0
        b = lax.shift_right_logical(u, 11) & (_NB - 1)
        plsc.addupdate_scatter(hist, [_R1 + lane * _NB + b], ones, mask=m)

    cand_scan(nv0, 0, r1_body)
    cand_scan(nv1, c1_base, r1_body)
    merge_round(sh1, _R1)
    b1, above1 = _find_bucket(merged, csum, rem1)
    p2 = (b0 << 10) | b1
    rem2 = rem1 - above1

    def r2_body(u):
        m = lax.shift_right_logical(u, 11) == p2
        b = lax.shift_right_logical(u, 1) & (_NB - 1)
        plsc.addupdate_scatter(hist, [_R2 + lane * _NB + b], ones, mask=m)

    cand_scan(nv0, 0, r2_body)
    cand_scan(nv1, c1_base, r2_body)
    merge_round(sh2, _R2)
    b2, above2 = _find_bucket(merged, csum, rem2)
    t0_bits = (((p2 << 10) | b2) << 1)
    t1_bits = t0_bits | 1
    rem3 = rem2 - above2

    def cnt_body(nv, cbase):
        @plsc.parallel_loop(0, nv, unroll=2, carry=(zeros16, zeros16))
        def accs(i, carry):
            a0, a1 = carry
            u = cand[pl.ds(cbase + i * _L, _L)]
            return (a0 + (u == t0_bits).astype(jnp.int32),
                    a1 + (u == t1_bits).astype(jnp.int32))
        a0, a1 = accs
        return jnp.sum(a0), jnp.sum(a1)

    n0_t0, n0_t1 = cnt_body(nv0, 0)
    n1_t0, n1_t1 = cnt_body(nv1, c1_base)
    row = jnp.where(lane == 0, n0_t0,
                    jnp.where(lane == 1, n1_t0,
                              jnp.where(lane == 2, n0_t1,
                                        jnp.where(lane == 3, n1_t1, 0))))
    merged[pl.ds(0, _L)] = row
    pltpu.sync_copy(merged.at[pl.ds(0, _L)], shcnt.at[pl.ds(sid * _L, _L)])
    plsc.subcore_barrier()
    pltpu.sync_copy(shcnt, hist.at[pl.ds(0, _NS * _L)])

    c_t1 = jnp.int32(0)
    for sp in range(_NS):
        rowv = hist[pl.ds(sp * _L, _L)]
        c_t1 = c_t1 + rowv[2] + rowv[3]
    use_t1 = rem3 <= c_t1
    t_bits = jnp.where(use_t1, t1_bits, t0_bits)
    e_keep = jnp.where(use_t1, rem3, rem3 - c_t1)

    h_own = 2 * sid + cid
    pre = jnp.int32(0)
    for sp in range(_NS):
        rowv = hist[pl.ds(sp * _L, _L)]
        a = jnp.where(use_t1, rowv[2], rowv[0])
        b = jnp.where(use_t1, rowv[3], rowv[1])
        pre = pre + jnp.where(2 * sp < h_own, a, 0)
        pre = pre + jnp.where(2 * sp + 1 < h_own, b, 0)
    n_own = jnp.where(cid == 0,
                      jnp.where(use_t1, n0_t1, n0_t0),
                      jnp.where(use_t1, n1_t1, n1_t0))
    quota = jnp.clip(e_keep - pre, 0, n_own)

    off = cid * half

    @pl.when(quota == n_own)
    def _():
        @plsc.parallel_loop(0, half // _L, unroll=8)
        def _(i):
            u = ubuf[pl.ds(off + i * _L, _L)]
            ubuf[pl.ds(off + i * _L, _L)] = jnp.where(u >= t_bits, u, 0)

    @pl.when(jnp.logical_and(quota == 0, n_own > 0))
    def _():
        @plsc.parallel_loop(0, half // _L, unroll=8)
        def _(i):
            u = ubuf[pl.ds(off + i * _L, _L)]
            ubuf[pl.ds(off + i * _L, _L)] = jnp.where(u > t_bits, u, 0)

    @pl.when(jnp.logical_and(quota > 0, quota < n_own))
    def _():
        def sb(i, running):
            u = ubuf[pl.ds(off + i * _L, _L)]
            eq = (u == t_bits).astype(jnp.int32)
            csum_v = plsc.cumsum(eq)
            rank = running + csum_v - eq
            keep = (u > t_bits) | ((eq == 1) & (rank < quota))
            ubuf[pl.ds(off + i * _L, _L)] = jnp.where(keep, u, 0)
            return running + csum_v[_L - 1]
        lax.fori_loop(0, half // _L, sb, jnp.int32(0))

    pltpu.sync_copy(ubuf.at[pl.ds(off, half)],
                    out_hbm.at[pl.ds(base + off, half)])


def _make_sc_call(n, kk):
    seg = n // _NS
    mesh = plsc.VectorSubcoreMesh(core_axis_name="c", subcore_axis_name="s",
                                  num_cores=_NC, num_subcores=_NS)
    return pl.kernel(
        functools.partial(_sc_body, n=n, kk=kk),
        out_type=jax.ShapeDtypeStruct((n,), jnp.int32),
        mesh=mesh,
        compiler_params=pltpu.CompilerParams(needs_layout_passes=False),
        scratch_types=[
            pltpu.VMEM((seg,), jnp.int32),
            pltpu.VMEM((3 * _L * _NB,), jnp.int32),
            pltpu.VMEM((_NB,), jnp.int32),
            pltpu.VMEM((seg + 2 * _L,), jnp.int32),
            pltpu.SMEM((_NB // _L,), jnp.int32),
            pltpu.VMEM_SHARED((_NS * _NB,), jnp.int32),
            pltpu.VMEM_SHARED((_NS * _NB,), jnp.int32),
            pltpu.VMEM_SHARED((_NS * _NB,), jnp.int32),
            pltpu.VMEM_SHARED((_NS * _L,), jnp.int32),
            pltpu.SemaphoreType.DMA,
        ],
    )


def kernel(x):
    n = x.size
    kk = _K * x.shape[0]
    xi = lax.bitcast_convert_type(x.reshape(-1), jnp.int32)
    out = _make_sc_call(n, kk)(xi)
    return lax.bitcast_convert_type(out, jnp.float32).reshape(x.shape)
